# baseline (speedup 1.0000x reference)
import functools

import jax
import jax.numpy as jnp
from jax import lax
from jax.experimental import pallas as pl
from jax.experimental.pallas import tpu as pltpu
from jax.experimental.pallas import tpu_sc as plsc

HID = 768
EPS = 1e-6
NC = 2
NS = 16
NW = NC * NS
GCHUNK = 64
TBLK = 256


@functools.cache
def _build_gather(n_tokens):
    rows_per_w = n_tokens // NW
    nchunks = rows_per_w // GCHUNK
    assert nchunks % 2 == 0
    mesh = plsc.VectorSubcoreMesh(core_axis_name="c", subcore_axis_name="s")

    @functools.partial(
        pl.kernel,
        mesh=mesh,
        out_type=jax.ShapeDtypeStruct((n_tokens, HID), jnp.float32),
        scratch_types=[
            pltpu.VMEM((rows_per_w,), jnp.int32),
            pltpu.VMEM((GCHUNK, HID), jnp.float32),
            pltpu.VMEM((GCHUNK, HID), jnp.float32),
            pltpu.SemaphoreType.DMA,
            pltpu.SemaphoreType.DMA,
            pltpu.SemaphoreType.DMA,
            pltpu.SemaphoreType.DMA,
        ],
    )
    def g(ids_hbm, word_hbm, out_hbm, idx_v, b0, b1, sg0, sg1, so0, so1):
        buf = (b0, b1)
        sg = (sg0, sg1)
        so = (so0, so1)
        wid = lax.axis_index("s") * NC + lax.axis_index("c")
        base = wid * rows_per_w

        pltpu.sync_copy(ids_hbm.at[pl.ds(base, rows_per_w)], idx_v)

        def gather(c, par):
            row0 = pl.multiple_of(c * GCHUNK, GCHUNK)
            return pltpu.make_async_copy(
                word_hbm.at[idx_v.at[pl.ds(row0, GCHUNK)]], buf[par], sg[par])

        def drain(c, par):
            row0 = pl.multiple_of(c * GCHUNK, GCHUNK)
            return pltpu.make_async_copy(
                buf[par], out_hbm.at[pl.ds(base + row0, GCHUNK)], so[par])

        gather(0, 0).start()
        gather(1, 1).start()

        def pair_body(c2, carry):
            c = c2 * 2
            gather(c, 0).wait()
            drain(c, 0).start()

            @pl.when(c + 2 < nchunks)
            def _refill0():
                drain(c, 0).wait()
                gather(c + 2, 0).start()

            gather(c + 1, 1).wait()
            drain(c + 1, 1).start()

            @pl.when(c + 3 < nchunks)
            def _refill1():
                drain(c + 1, 1).wait()
                gather(c + 3, 1).start()

            return carry

        lax.fori_loop(0, nchunks // 2, pair_body, 0)
        drain(nchunks - 2, 0).wait()
        drain(nchunks - 1, 1).wait()

    return g


def _ln_body(x_ref, pos_ref, g_ref, b_ref, o_ref):
    x = x_ref[...] + pos_ref[...][None]
    mean = jnp.mean(x, axis=-1, keepdims=True)
    msq = jnp.mean(x * x, axis=-1, keepdims=True)
    var = msq - mean * mean
    o_ref[...] = (x - mean) * lax.rsqrt(var + EPS) * g_ref[...] + b_ref[...]


def _ln_body_acc(acc_ref, x_ref, pos_ref, g_ref, b_ref, o_ref):
    del acc_ref
    _ln_body(x_ref, pos_ref, g_ref, b_ref, o_ref)


@functools.cache
def _build_ln_pair(total_b, half_b, seq):
    common_in = [
        pl.BlockSpec((half_b, TBLK, HID), lambda j: (0, j, 0)),
        pl.BlockSpec((TBLK, HID), lambda j: (j, 0)),
        pl.BlockSpec((HID,), lambda j: (0,)),
        pl.BlockSpec((HID,), lambda j: (0,)),
    ]
    out_shape = jax.ShapeDtypeStruct((total_b, seq, HID), jnp.float32)
    ln0 = pl.pallas_call(
        _ln_body,
        grid=(seq // TBLK,),
        in_specs=common_in,
        out_specs=pl.BlockSpec((half_b, TBLK, HID), lambda j: (0, j, 0)),
        out_shape=out_shape,
    )
    ln1 = pl.pallas_call(
        _ln_body_acc,
        grid=(seq // TBLK,),
        in_specs=[pl.BlockSpec(memory_space=pl.ANY)] + common_in,
        out_specs=pl.BlockSpec((half_b, TBLK, HID), lambda j: (1, j, 0)),
        out_shape=out_shape,
        input_output_aliases={0: 0},
    )
    return ln0, ln1


def kernel(input_ids, word_embeddings, position_embeddings, gamma, beta):
    b, s = input_ids.shape
    hb = b // 2
    ids0 = input_ids[:hb].reshape(-1).astype(jnp.int32)
    ids1 = input_ids[hb:].reshape(-1).astype(jnp.int32)
    g0 = _build_gather(hb * s)(ids0, word_embeddings)
    g1 = _build_gather(hb * s)(ids1, word_embeddings)
    ln0, ln1 = _build_ln_pair(b, hb, s)
    out = ln0(g0.reshape(hb, s, HID), position_embeddings, gamma, beta)
    out = ln1(out, g1.reshape(hb, s, HID), position_embeddings, gamma, beta)
    return out
